# baseline (speedup 1.0000x reference)
import functools

import jax
import jax.numpy as jnp
from jax import lax
from jax.experimental import pallas as pl
from jax.experimental.pallas import tpu as pltpu
from jax.experimental.pallas import tpu_sc as plsc

_DIM = 64
_IW = 128
_G = 5


@functools.lru_cache(maxsize=None)
def _gather_call(n_idx_rows):
    info = plsc.get_sparse_core_info()
    nc, ns = info.num_cores, info.num_subcores
    nw = nc * ns
    rows_per_w = n_idx_rows // nw
    n_chunks = rows_per_w // _G
    assert rows_per_w % _G == 0 and n_chunks % 2 == 0

    mesh = plsc.VectorSubcoreMesh(core_axis_name="c", subcore_axis_name="s")

    @functools.partial(
        pl.kernel,
        out_type=jax.ShapeDtypeStruct((n_idx_rows, _IW, _DIM), jnp.float32),
        mesh=mesh,
        scratch_types=[
            pltpu.VMEM((rows_per_w, _IW), jnp.int32),
            pltpu.VMEM((2, _G, _IW, _DIM), jnp.float32),
            pltpu.SemaphoreType.DMA,
            pltpu.SemaphoreType.DMA,
            pltpu.SemaphoreType.DMA,
            pltpu.SemaphoreType.DMA,
        ],
        compiler_params=pltpu.CompilerParams(use_tc_tiling_on_sc=False),
    )
    def k(idx_hbm, table_hbm, out_hbm, idx_v, rows_v, sg0, sg1, so0, so1):
        wid = lax.axis_index("s") * nc + lax.axis_index("c")
        base = wid * rows_per_w
        pltpu.sync_copy(idx_hbm.at[pl.ds(base, rows_per_w)], idx_v)

        sg = (sg0, sg1)
        so = (so0, so1)

        def issue_gather(g, s):
            for j in range(_G):
                pltpu.async_copy(
                    table_hbm.at[idx_v.at[g * _G + j]], rows_v.at[s, j], sg[s])

        def wait_gather(s):
            for j in range(_G):
                pltpu.make_async_copy(
                    table_hbm.at[idx_v.at[j]], rows_v.at[s, j], sg[s]).wait()

        def issue_store(g, s):
            pltpu.async_copy(
                rows_v.at[s], out_hbm.at[pl.ds(base + g * _G, _G)], so[s])

        def wait_store(s):
            pltpu.make_async_copy(
                rows_v.at[s], out_hbm.at[pl.ds(base, _G)], so[s]).wait()

        issue_gather(0, 0)

        def pair(p, carry):
            g0 = 2 * p

            @pl.when(p > 0)
            def _():
                wait_store(1)

            issue_gather(g0 + 1, 1)
            wait_gather(0)
            issue_store(g0, 0)

            @pl.when(p + 1 < n_chunks // 2)
            def _():
                wait_store(0)
                issue_gather(g0 + 2, 0)

            wait_gather(1)
            issue_store(g0 + 1, 1)
            return carry

        lax.fori_loop(0, n_chunks // 2, pair, 0)
        wait_store(0)
        wait_store(1)

    return k


def kernel(inputs, table):
    n_idx = inputs.shape[0] * inputs.shape[1]
    idx = inputs.reshape(n_idx // _IW, _IW).astype(jnp.int32)
    out = _gather_call(n_idx // _IW)(idx, table)
    return out.reshape(inputs.shape + (table.shape[1],))


# speedup vs baseline: 1.0010x; 1.0010x over previous
import functools

import jax
import jax.numpy as jnp
from jax import lax
from jax.experimental import pallas as pl
from jax.experimental.pallas import tpu as pltpu
from jax.experimental.pallas import tpu_sc as plsc

_G = 4


@functools.lru_cache(maxsize=None)
def _gather_call(b, l, d):
    info = plsc.get_sparse_core_info()
    nc, ns = info.num_cores, info.num_subcores
    nw = nc * ns
    b_per_w = b // nw
    n_chunks = b_per_w // _G
    assert b % nw == 0 and b_per_w % _G == 0 and n_chunks % 2 == 0

    mesh = plsc.VectorSubcoreMesh(core_axis_name="c", subcore_axis_name="s")

    @functools.partial(
        pl.kernel,
        out_type=jax.ShapeDtypeStruct((b, l, d), jnp.float32),
        mesh=mesh,
        scratch_types=[
            pltpu.VMEM((b_per_w, l), jnp.int32),
            pltpu.VMEM((2, _G, l, d), jnp.float32),
            pltpu.SemaphoreType.DMA,
            pltpu.SemaphoreType.DMA,
            pltpu.SemaphoreType.DMA,
            pltpu.SemaphoreType.DMA,
        ],
        compiler_params=pltpu.CompilerParams(use_tc_tiling_on_sc=False),
    )
    def k(idx_hbm, table_hbm, out_hbm, idx_v, rows_v, sg0, sg1, so0, so1):
        wid = lax.axis_index("s") * nc + lax.axis_index("c")
        base = wid * b_per_w
        pltpu.sync_copy(idx_hbm.at[pl.ds(base, b_per_w)], idx_v)

        sg = (sg0, sg1)
        so = (so0, so1)

        def issue_gather(g, s):
            for j in range(_G):
                pltpu.async_copy(
                    table_hbm.at[idx_v.at[g * _G + j]], rows_v.at[s, j], sg[s])

        def wait_gather(s):
            for j in range(_G):
                pltpu.make_async_copy(
                    table_hbm.at[idx_v.at[j]], rows_v.at[s, j], sg[s]).wait()

        def issue_store(g, s):
            pltpu.async_copy(
                rows_v.at[s], out_hbm.at[pl.ds(base + g * _G, _G)], so[s])

        def wait_store(s):
            pltpu.make_async_copy(
                rows_v.at[s], out_hbm.at[pl.ds(base, _G)], so[s]).wait()

        issue_gather(0, 0)

        def pair(p, carry):
            g0 = 2 * p

            @pl.when(p > 0)
            def _():
                wait_store(1)

            issue_gather(g0 + 1, 1)
            wait_gather(0)
            issue_store(g0, 0)

            @pl.when(p + 1 < n_chunks // 2)
            def _():
                wait_store(0)
                issue_gather(g0 + 2, 0)

            wait_gather(1)
            issue_store(g0 + 1, 1)
            return carry

        lax.fori_loop(0, n_chunks // 2, pair, 0)
        wait_store(0)
        wait_store(1)

    return k


def kernel(inputs, table):
    b, l = inputs.shape
    return _gather_call(b, l, table.shape[1])(inputs.astype(jnp.int32), table)
